# Initial kernel scaffold; baseline (speedup 1.0000x reference)
import jax
import jax.numpy as jnp
from jax.experimental import pallas as pl


def kernel(x, edge_index, W, att_src, att_dst, bias):
    raise NotImplementedError("write your pallas kernel here")


# speedup vs baseline: 8.3921x; 8.3921x over previous
import functools

import jax
import jax.numpy as jnp
from jax import lax
from jax.experimental import pallas as pl
from jax.experimental.pallas import tpu as pltpu
from jax.experimental.pallas import tpu_sc as plsc

NEG_BIG = -1e30
SUB = 64
N_SUBCORES = 16
N_CORES = 2


def _proj_body(x_ref, w_ref, asv_ref, adv_ref, h_ref, as_ref, ad_ref, ms_ref, md_ref):
    i = pl.program_id(0)
    h = jnp.dot(x_ref[...], w_ref[...], preferred_element_type=jnp.float32)
    h_ref[...] = h
    a_s = jnp.sum(h * asv_ref[...], axis=1, keepdims=True)
    a_d = jnp.sum(h * adv_ref[...], axis=1, keepdims=True)
    as_ref[...] = a_s
    ad_ref[...] = a_d
    bs = jnp.max(a_s, keepdims=True)
    bd = jnp.max(a_d, keepdims=True)
    ms_ref[...] = jnp.where(i == 0, bs, jnp.maximum(ms_ref[...], bs))
    md_ref[...] = jnp.where(i == 0, bd, jnp.maximum(md_ref[...], bd))


def _project(xr, W, att_src, att_dst):
    BN, F = xr.shape
    RB = 1000
    grid = BN // RB
    return pl.pallas_call(
        _proj_body,
        grid=(grid,),
        in_specs=[
            pl.BlockSpec((RB, F), lambda i: (i, 0)),
            pl.BlockSpec((F, F), lambda i: (0, 0)),
            pl.BlockSpec((1, F), lambda i: (0, 0)),
            pl.BlockSpec((1, F), lambda i: (0, 0)),
        ],
        out_specs=[
            pl.BlockSpec((RB, F), lambda i: (i, 0)),
            pl.BlockSpec((RB, 1), lambda i: (i, 0)),
            pl.BlockSpec((RB, 1), lambda i: (i, 0)),
            pl.BlockSpec((1, 1), lambda i: (0, 0)),
            pl.BlockSpec((1, 1), lambda i: (0, 0)),
        ],
        out_shape=[
            jax.ShapeDtypeStruct((BN, F), jnp.float32),
            jax.ShapeDtypeStruct((BN, 1), jnp.float32),
            jax.ShapeDtypeStruct((BN, 1), jnp.float32),
            jax.ShapeDtypeStruct((1, 1), jnp.float32),
            jax.ShapeDtypeStruct((1, 1), jnp.float32),
        ],
    )(xr, W, att_src.reshape(1, F), att_dst.reshape(1, F))


def _edge_body(N, NP, F, W, NW,
               h_hbm, asrc_hbm, adst_hbm, src_hbm, dst_hbm, c16_hbm, bias_hbm,
               out_hbm,
               asrc_v, adst_v, srcw, dstw, pend_g, pend_d, pend_e, hrows,
               acc, den2, outb, cbuf, bias_v, sem):
    c = lax.axis_index("c")
    s = lax.axis_index("s")
    coff = c * NP
    NBLK = N // 16
    KMAX = (NBLK + N_SUBCORES - 1) // N_SUBCORES
    SUBC = SUB // 16

    pltpu.sync_copy(asrc_hbm.at[pl.ds(coff * 1, NP)], asrc_v)
    pltpu.sync_copy(adst_hbm.at[pl.ds(coff * 1, NP)], adst_v)
    pltpu.sync_copy(c16_hbm, cbuf)
    pltpu.sync_copy(bias_hbm, bias_v)
    c16 = cbuf[...]

    zeros16 = jnp.zeros((16,), jnp.float32)

    def _zacc(r, carry):
        for q in range(F // 16):
            acc[pl.ds(r * F + q * 16, 16)] = zeros16
        den2[pl.ds(r * 16, 16)] = zeros16
        return carry

    lax.fori_loop(0, 16 * KMAX, _zacc, 0)

    i16 = lax.iota(jnp.int32, 16)

    def _window(w, carry):
        pltpu.sync_copy(src_hbm.at[pl.ds(w * W, W)], srcw)
        pltpu.sync_copy(dst_hbm.at[pl.ds(w * W, W)], dstw)

        def _scan(j, cnt):
            s16 = srcw[pl.ds(j * 16, 16)]
            d16 = dstw[pl.ds(j * 16, 16)]
            a_s = plsc.load_gather(asrc_v, [s16])
            a_d = plsc.load_gather(adst_v, [d16])
            z = a_s + a_d
            al = jnp.where(z >= 0.0, z, z * 0.2)
            e = jnp.exp(al - c16)
            blk = d16 >> 4
            m = (blk & 15) == s
            d_l = ((d16 >> 8) << 4) | (d16 & 15)
            plsc.store_compressed(pend_g.at[pl.ds(cnt, 16)], s16 + coff, mask=m)
            plsc.store_compressed(pend_d.at[pl.ds(cnt, 16)], d_l, mask=m)
            plsc.store_compressed(pend_e.at[pl.ds(cnt, 16)], e, mask=m)
            return cnt + plsc.all_reduce_population_count(m)[0]

        cnt = lax.fori_loop(0, W // 16, _scan, jnp.int32(0))

        k_pad = (SUB - (cnt % SUB)) % SUB
        for i in range(SUB // 16):
            @pl.when(i * 16 < k_pad)
            def _pad():
                pend_g[pl.ds(cnt + i * 16, 16)] = coff + N + i16
                pend_d[pl.ds(cnt + i * 16, 16)] = i16
                pend_e[pl.ds(cnt + i * 16, 16)] = zeros16
        cnt_p = cnt + k_pad

        def _chunk(k, _2):
            pltpu.async_copy(h_hbm.at[pend_g.at[pl.ds(k * SUB, SUB)]],
                             hrows, sem).wait()

            def _grp(g, _3):
                pd16 = pend_d[pl.ds(k * SUB + g * 16, 16)]
                pe16 = pend_e[pl.ds(k * SUB + g * 16, 16)]
                for rr in range(16):
                    r = g * 16 + rr
                    d_s = pd16[rr]
                    e_b = pe16[jnp.full((16,), rr, jnp.int32)]
                    for q in range(F // 16):
                        acc[pl.ds(d_s * F + q * 16, 16)] = (
                            acc[pl.ds(d_s * F + q * 16, 16)]
                            + hrows[r, pl.ds(q * 16, 16)] * e_b)
                    den2[pl.ds(d_s * 16, 16)] = den2[pl.ds(d_s * 16, 16)] + e_b
                return _3

            lax.fori_loop(0, SUBC, _grp, 0)
            return _2

        lax.fori_loop(0, cnt_p // SUB, _chunk, 0)
        return carry

    lax.fori_loop(0, NW, _window, 0)

    for k in range(KMAX):
        blk = s + N_SUBCORES * k

        @pl.when(blk < NBLK)
        def _out():
            def _erow(rr, _2):
                r = k * 16 + rr
                inv = 1.0 / (den2[pl.ds(r * 16, 16)] + 1e-16)
                for q in range(F // 16):
                    outb[rr, pl.ds(q * 16, 16)] = (
                        acc[pl.ds(r * F + q * 16, 16)] * inv
                        + bias_v[pl.ds(q * 16, 16)])
                return _2

            lax.fori_loop(0, 16, _erow, 0)
            pltpu.sync_copy(outb, out_hbm.at[pl.ds(c * N + blk * 16, 16)])


def _edge_pass(h_pad, asrc_pad, adst_pad, src_pad, dst_pad, c16, bias, N, NP, F,
               W, NW):
    BN = (h_pad.shape[0] // NP) * N
    PEND = W + 16
    mesh = plsc.VectorSubcoreMesh(core_axis_name="c", subcore_axis_name="s")
    body = functools.partial(_edge_body, N, NP, F, W, NW)
    return pl.kernel(
        body,
        out_type=jax.ShapeDtypeStruct((BN, F), jnp.float32),
        mesh=mesh,
        compiler_params=pltpu.CompilerParams(needs_layout_passes=False),
        scratch_types=[
            pltpu.VMEM((NP,), jnp.float32),
            pltpu.VMEM((NP,), jnp.float32),
            pltpu.VMEM((W,), jnp.int32),
            pltpu.VMEM((W,), jnp.int32),
            pltpu.VMEM((PEND,), jnp.int32),
            pltpu.VMEM((PEND,), jnp.int32),
            pltpu.VMEM((PEND,), jnp.float32),
            pltpu.VMEM((SUB, F), jnp.float32),
            pltpu.VMEM((640 * F,), jnp.float32),
            pltpu.VMEM((640 * 16,), jnp.float32),
            pltpu.VMEM((16, F), jnp.float32),
            pltpu.VMEM((16,), jnp.float32),
            pltpu.VMEM((F,), jnp.float32),
            pltpu.SemaphoreType.DMA,
        ],
    )(h_pad, asrc_pad, adst_pad, src_pad, dst_pad, c16, bias)


def kernel(x, edge_index, W, att_src, att_dst, bias):
    B, N, F = x.shape
    E = edge_index.shape[1]
    BN = B * N
    NP = N + 16

    h, asrc, adst, ms, md = _project(x.reshape(BN, F), W, att_src, att_dst)

    sent_a = jnp.full((16,), NEG_BIG, jnp.float32)
    sent_h = jnp.zeros((16, F), jnp.float32)
    h_parts, as_parts, ad_parts = [], [], []
    for b in range(B):
        h_parts += [h[b * N:(b + 1) * N], sent_h]
        as_parts += [asrc[b * N:(b + 1) * N, 0], sent_a]
        ad_parts += [adst[b * N:(b + 1) * N, 0], sent_a]
    h_pad = jnp.concatenate(h_parts, axis=0)
    asrc_pad = jnp.concatenate(as_parts)
    adst_pad = jnp.concatenate(ad_parts)

    EN = E + N
    W = 1536
    NW = (EN + W - 1) // W
    ET = NW * W
    P = ET - EN
    loop_ids = jnp.arange(N, dtype=jnp.int32)
    pad_ids = jnp.arange(P, dtype=jnp.int32)
    src_pad = jnp.concatenate([edge_index[0].astype(jnp.int32), loop_ids,
                               N + (pad_ids % 16)])
    dst_pad = jnp.concatenate([edge_index[1].astype(jnp.int32), loop_ids,
                               jnp.full((P,), N, jnp.int32)])

    z0 = ms[0, 0] + md[0, 0]
    cb = jnp.where(z0 >= 0.0, z0, 0.2 * z0)
    c16 = jnp.broadcast_to(cb, (16,)).astype(jnp.float32)

    out = _edge_pass(h_pad, asrc_pad, adst_pad, src_pad, dst_pad, c16, bias,
                     N, NP, F, W, NW)
    return out.reshape(B, N, F)


# speedup vs baseline: 9.8950x; 1.1791x over previous
import functools

import jax
import jax.numpy as jnp
from jax import lax
from jax.experimental import pallas as pl
from jax.experimental.pallas import tpu as pltpu
from jax.experimental.pallas import tpu_sc as plsc

NEG_BIG = -1e30
SUB = 64
N_SUBCORES = 16
N_CORES = 2


def _proj_body(x_ref, w_ref, asv_ref, adv_ref, h_ref, as_ref, ad_ref, ms_ref, md_ref):
    i = pl.program_id(0)
    h = jnp.dot(x_ref[...], w_ref[...], preferred_element_type=jnp.float32)
    h_ref[...] = h
    a_s = jnp.sum(h * asv_ref[...], axis=1, keepdims=True)
    a_d = jnp.sum(h * adv_ref[...], axis=1, keepdims=True)
    as_ref[...] = a_s
    ad_ref[...] = a_d
    bs = jnp.max(a_s, keepdims=True)
    bd = jnp.max(a_d, keepdims=True)
    ms_ref[...] = jnp.where(i == 0, bs, jnp.maximum(ms_ref[...], bs))
    md_ref[...] = jnp.where(i == 0, bd, jnp.maximum(md_ref[...], bd))


def _project(xr, W, att_src, att_dst):
    BN, F = xr.shape
    RB = 1000
    grid = BN // RB
    return pl.pallas_call(
        _proj_body,
        grid=(grid,),
        in_specs=[
            pl.BlockSpec((RB, F), lambda i: (i, 0)),
            pl.BlockSpec((F, F), lambda i: (0, 0)),
            pl.BlockSpec((1, F), lambda i: (0, 0)),
            pl.BlockSpec((1, F), lambda i: (0, 0)),
        ],
        out_specs=[
            pl.BlockSpec((RB, F), lambda i: (i, 0)),
            pl.BlockSpec((RB, 1), lambda i: (i, 0)),
            pl.BlockSpec((RB, 1), lambda i: (i, 0)),
            pl.BlockSpec((1, 1), lambda i: (0, 0)),
            pl.BlockSpec((1, 1), lambda i: (0, 0)),
        ],
        out_shape=[
            jax.ShapeDtypeStruct((BN, F), jnp.float32),
            jax.ShapeDtypeStruct((BN, 1), jnp.float32),
            jax.ShapeDtypeStruct((BN, 1), jnp.float32),
            jax.ShapeDtypeStruct((1, 1), jnp.float32),
            jax.ShapeDtypeStruct((1, 1), jnp.float32),
        ],
    )(xr, W, att_src.reshape(1, F), att_dst.reshape(1, F))


def _edge_body(N, NP, F, W, NW,
               h_hbm, asrc_hbm, adst_hbm, src_hbm, dst_hbm, c16_hbm, bias_hbm,
               out_hbm,
               asrc_v, adst_v, srcw, dstw, pend_g, pend_d, pend_e, hrows,
               acc, den2, outb, cbuf, bias_v, sem):
    c = lax.axis_index("c")
    s = lax.axis_index("s")
    coff = c * NP
    NBLK = N // 16
    KMAX = (NBLK + N_SUBCORES - 1) // N_SUBCORES
    SUBC = SUB // 16

    pltpu.sync_copy(asrc_hbm.at[pl.ds(coff * 1, NP)], asrc_v)
    pltpu.sync_copy(adst_hbm.at[pl.ds(coff * 1, NP)], adst_v)
    pltpu.sync_copy(c16_hbm, cbuf)
    pltpu.sync_copy(bias_hbm, bias_v)
    c16 = cbuf[...]

    zeros16 = jnp.zeros((16,), jnp.float32)

    def _zacc(r, carry):
        for q in range(F // 16):
            acc[pl.ds(r * F + q * 16, 16)] = zeros16
        den2[pl.ds(r * 16, 16)] = zeros16
        return carry

    lax.fori_loop(0, 16 * KMAX, _zacc, 0)

    i16 = lax.iota(jnp.int32, 16)

    def _chunk(k, _2):
        pltpu.async_copy(h_hbm.at[pend_g.at[pl.ds(k * SUB, SUB)]],
                         hrows, sem).wait()

        def _grp(g, _3):
            pd16 = pend_d[pl.ds(k * SUB + g * 16, 16)]
            pe16 = pend_e[pl.ds(k * SUB + g * 16, 16)]
            for rr in range(16):
                r = g * 16 + rr
                d_s = pd16[rr]
                e_b = pe16[jnp.full((16,), rr, jnp.int32)]
                for q in range(F // 16):
                    acc[pl.ds(d_s * F + q * 16, 16)] = (
                        acc[pl.ds(d_s * F + q * 16, 16)]
                        + hrows[r, pl.ds(q * 16, 16)] * e_b)
                den2[pl.ds(d_s * 16, 16)] = den2[pl.ds(d_s * 16, 16)] + e_b
            return _3

        lax.fori_loop(0, SUBC, _grp, 0)
        return _2

    def _window(w, rem):
        pltpu.sync_copy(src_hbm.at[pl.ds(w * W, W)], srcw)
        pltpu.sync_copy(dst_hbm.at[pl.ds(w * W, W)], dstw)

        def _scan(j, cnt):
            s16 = srcw[pl.ds(j * 16, 16)]
            d16 = dstw[pl.ds(j * 16, 16)]
            a_s = plsc.load_gather(asrc_v, [s16])
            a_d = plsc.load_gather(adst_v, [d16])
            z = a_s + a_d
            al = jnp.where(z >= 0.0, z, z * 0.2)
            e = jnp.exp(al - c16)
            blk = d16 >> 4
            m = (blk & 15) == s
            d_l = ((d16 >> 8) << 4) | (d16 & 15)
            plsc.store_compressed(pend_g.at[pl.ds(cnt, 16)], s16 + coff, mask=m)
            plsc.store_compressed(pend_d.at[pl.ds(cnt, 16)], d_l, mask=m)
            plsc.store_compressed(pend_e.at[pl.ds(cnt, 16)], e, mask=m)
            return cnt + plsc.all_reduce_population_count(m)[0]

        cnt = lax.fori_loop(0, W // 16, _scan, rem)

        n_full = cnt // SUB
        lax.fori_loop(0, n_full, _chunk, 0)
        for i in range(SUB // 16):
            pend_g[pl.ds(i * 16, 16)] = pend_g[pl.ds(n_full * SUB + i * 16, 16)]
            pend_d[pl.ds(i * 16, 16)] = pend_d[pl.ds(n_full * SUB + i * 16, 16)]
            pend_e[pl.ds(i * 16, 16)] = pend_e[pl.ds(n_full * SUB + i * 16, 16)]
        return cnt - n_full * SUB

    rem = lax.fori_loop(0, NW, _window, jnp.int32(0))

    k_pad = (SUB - (rem % SUB)) % SUB
    for i in range(SUB // 16):
        @pl.when(i * 16 < k_pad)
        def _pad():
            pend_g[pl.ds(rem + i * 16, 16)] = coff + N + i16
            pend_d[pl.ds(rem + i * 16, 16)] = i16
            pend_e[pl.ds(rem + i * 16, 16)] = zeros16

    @pl.when(rem > 0)
    def _tail():
        _chunk(0, 0)

    for k in range(KMAX):
        blk = s + N_SUBCORES * k

        @pl.when(blk < NBLK)
        def _out():
            def _erow(rr, _2):
                r = k * 16 + rr
                inv = 1.0 / (den2[pl.ds(r * 16, 16)] + 1e-16)
                for q in range(F // 16):
                    outb[rr, pl.ds(q * 16, 16)] = (
                        acc[pl.ds(r * F + q * 16, 16)] * inv
                        + bias_v[pl.ds(q * 16, 16)])
                return _2

            lax.fori_loop(0, 16, _erow, 0)
            pltpu.sync_copy(outb, out_hbm.at[pl.ds(c * N + blk * 16, 16)])


def _edge_pass(h_pad, asrc_pad, adst_pad, src_pad, dst_pad, c16, bias, N, NP, F,
               W, NW):
    BN = (h_pad.shape[0] // NP) * N
    PEND = W + SUB + 16
    mesh = plsc.VectorSubcoreMesh(core_axis_name="c", subcore_axis_name="s")
    body = functools.partial(_edge_body, N, NP, F, W, NW)
    return pl.kernel(
        body,
        out_type=jax.ShapeDtypeStruct((BN, F), jnp.float32),
        mesh=mesh,
        compiler_params=pltpu.CompilerParams(needs_layout_passes=False),
        scratch_types=[
            pltpu.VMEM((NP,), jnp.float32),
            pltpu.VMEM((NP,), jnp.float32),
            pltpu.VMEM((W,), jnp.int32),
            pltpu.VMEM((W,), jnp.int32),
            pltpu.VMEM((PEND,), jnp.int32),
            pltpu.VMEM((PEND,), jnp.int32),
            pltpu.VMEM((PEND,), jnp.float32),
            pltpu.VMEM((SUB, F), jnp.float32),
            pltpu.VMEM((640 * F,), jnp.float32),
            pltpu.VMEM((640 * 16,), jnp.float32),
            pltpu.VMEM((16, F), jnp.float32),
            pltpu.VMEM((16,), jnp.float32),
            pltpu.VMEM((F,), jnp.float32),
            pltpu.SemaphoreType.DMA,
        ],
    )(h_pad, asrc_pad, adst_pad, src_pad, dst_pad, c16, bias)


def kernel(x, edge_index, W, att_src, att_dst, bias):
    B, N, F = x.shape
    E = edge_index.shape[1]
    BN = B * N
    NP = N + 16

    h, asrc, adst, ms, md = _project(x.reshape(BN, F), W, att_src, att_dst)

    sent_a = jnp.full((16,), NEG_BIG, jnp.float32)
    sent_h = jnp.zeros((16, F), jnp.float32)
    h_parts, as_parts, ad_parts = [], [], []
    for b in range(B):
        h_parts += [h[b * N:(b + 1) * N], sent_h]
        as_parts += [asrc[b * N:(b + 1) * N, 0], sent_a]
        ad_parts += [adst[b * N:(b + 1) * N, 0], sent_a]
    h_pad = jnp.concatenate(h_parts, axis=0)
    asrc_pad = jnp.concatenate(as_parts)
    adst_pad = jnp.concatenate(ad_parts)

    EN = E + N
    W = 1536
    NW = (EN + W - 1) // W
    ET = NW * W
    P = ET - EN
    loop_ids = jnp.arange(N, dtype=jnp.int32)
    pad_ids = jnp.arange(P, dtype=jnp.int32)
    src_pad = jnp.concatenate([edge_index[0].astype(jnp.int32), loop_ids,
                               N + (pad_ids % 16)])
    dst_pad = jnp.concatenate([edge_index[1].astype(jnp.int32), loop_ids,
                               jnp.full((P,), N, jnp.int32)])

    z0 = ms[0, 0] + md[0, 0]
    cb = jnp.where(z0 >= 0.0, z0, 0.2 * z0)
    c16 = jnp.broadcast_to(cb, (16,)).astype(jnp.float32)

    out = _edge_pass(h_pad, asrc_pad, adst_pad, src_pad, dst_pad, c16, bias,
                     N, NP, F, W, NW)
    return out.reshape(B, N, F)


# speedup vs baseline: 9.9437x; 1.0049x over previous
import functools

import jax
import jax.numpy as jnp
from jax import lax
from jax.experimental import pallas as pl
from jax.experimental.pallas import tpu as pltpu
from jax.experimental.pallas import tpu_sc as plsc

NEG_BIG = -1e30
SUB = 48
N_SUBCORES = 16
N_CORES = 2


def _proj_body(x_ref, w_ref, asv_ref, adv_ref, h_ref, as_ref, ad_ref, ms_ref, md_ref):
    i = pl.program_id(0)
    h = jnp.dot(x_ref[...], w_ref[...], preferred_element_type=jnp.float32)
    h_ref[...] = h
    a_s = jnp.sum(h * asv_ref[...], axis=1, keepdims=True)
    a_d = jnp.sum(h * adv_ref[...], axis=1, keepdims=True)
    as_ref[...] = a_s
    ad_ref[...] = a_d
    bs = jnp.max(a_s, keepdims=True)
    bd = jnp.max(a_d, keepdims=True)
    ms_ref[...] = jnp.where(i == 0, bs, jnp.maximum(ms_ref[...], bs))
    md_ref[...] = jnp.where(i == 0, bd, jnp.maximum(md_ref[...], bd))


def _project(xr, W, att_src, att_dst):
    BN, F = xr.shape
    RB = 1000
    grid = BN // RB
    return pl.pallas_call(
        _proj_body,
        grid=(grid,),
        in_specs=[
            pl.BlockSpec((RB, F), lambda i: (i, 0)),
            pl.BlockSpec((F, F), lambda i: (0, 0)),
            pl.BlockSpec((1, F), lambda i: (0, 0)),
            pl.BlockSpec((1, F), lambda i: (0, 0)),
        ],
        out_specs=[
            pl.BlockSpec((RB, F), lambda i: (i, 0)),
            pl.BlockSpec((RB, 1), lambda i: (i, 0)),
            pl.BlockSpec((RB, 1), lambda i: (i, 0)),
            pl.BlockSpec((1, 1), lambda i: (0, 0)),
            pl.BlockSpec((1, 1), lambda i: (0, 0)),
        ],
        out_shape=[
            jax.ShapeDtypeStruct((BN, F), jnp.float32),
            jax.ShapeDtypeStruct((BN, 1), jnp.float32),
            jax.ShapeDtypeStruct((BN, 1), jnp.float32),
            jax.ShapeDtypeStruct((1, 1), jnp.float32),
            jax.ShapeDtypeStruct((1, 1), jnp.float32),
        ],
    )(xr, W, att_src.reshape(1, F), att_dst.reshape(1, F))


def _edge_body(N, NP, F, W, NW,
               h_hbm, asrc_hbm, adst_hbm, sd_hbm, c16_hbm, bias_hbm,
               out_hbm,
               asrc_v, adst_v, sdw, sdw2, pend_g, pend_d, pend_e, hrows,
               acc, den2, outb, cbuf, bias_v, sem, semS):
    c = lax.axis_index("c")
    s = lax.axis_index("s")
    coff = c * NP
    NBLK = N // 16
    KMAX = (NBLK + N_SUBCORES - 1) // N_SUBCORES
    SUBC = SUB // 16

    pltpu.sync_copy(asrc_hbm.at[pl.ds(coff * 1, NP)], asrc_v)
    pltpu.sync_copy(adst_hbm.at[pl.ds(coff * 1, NP)], adst_v)
    pltpu.sync_copy(c16_hbm, cbuf)
    pltpu.sync_copy(bias_hbm, bias_v)
    c16 = cbuf[...]

    zeros16 = jnp.zeros((16,), jnp.float32)

    def _zacc(r, carry):
        for q in range(F // 16):
            acc[pl.ds(r * F + q * 16, 16)] = zeros16
        den2[pl.ds(r * 16, 16)] = zeros16
        return carry

    lax.fori_loop(0, 16 * KMAX, _zacc, 0)

    i16 = lax.iota(jnp.int32, 16)

    def _fire_sd(w_):
        pltpu.async_copy(sd_hbm.at[pl.ds(w_ * (2 * W), 2 * W)], sdw2, semS)

    _fire_sd(0)

    def _chunk(k, _2):
        pltpu.async_copy(h_hbm.at[pend_g.at[pl.ds(k * SUB, SUB)]],
                         hrows, sem).wait()

        def _grp(g, _3):
            pd16 = pend_d[pl.ds(k * SUB + g * 16, 16)]
            pe16 = pend_e[pl.ds(k * SUB + g * 16, 16)]
            for rr in range(16):
                r = g * 16 + rr
                d_s = pd16[rr]
                e_b = pe16[jnp.full((16,), rr, jnp.int32)]
                for q in range(F // 16):
                    acc[pl.ds(d_s * F + q * 16, 16)] = (
                        acc[pl.ds(d_s * F + q * 16, 16)]
                        + hrows[r, pl.ds(q * 16, 16)] * e_b)
                den2[pl.ds(d_s * 16, 16)] = den2[pl.ds(d_s * 16, 16)] + e_b
            return _3

        lax.fori_loop(0, SUBC, _grp, 0)
        return _2

    def _window(w, rem):
        pltpu.make_async_copy(sd_hbm.at[pl.ds(0, 2 * W)], sdw2, semS).wait()

        def _cp(j, carry):
            sdw[pl.ds(j * 16, 16)] = sdw2[pl.ds(j * 16, 16)]
            return carry

        lax.fori_loop(0, (2 * W) // 16, _cp, 0)

        @pl.when(w + 1 < NW)
        def _pf():
            _fire_sd(w + 1)

        def _scan(j, cnt):
            s16 = sdw[pl.ds(j * 16, 16)]
            d16 = sdw[pl.ds(W + j * 16, 16)]
            a_s = plsc.load_gather(asrc_v, [s16])
            a_d = plsc.load_gather(adst_v, [d16])
            z = a_s + a_d
            al = jnp.where(z >= 0.0, z, z * 0.2)
            e = jnp.exp(al - c16)
            blk = d16 >> 4
            m = (blk & 15) == s
            d_l = ((d16 >> 8) << 4) | (d16 & 15)
            plsc.store_compressed(pend_g.at[pl.ds(cnt, 16)], s16 + coff, mask=m)
            plsc.store_compressed(pend_d.at[pl.ds(cnt, 16)], d_l, mask=m)
            plsc.store_compressed(pend_e.at[pl.ds(cnt, 16)], e, mask=m)
            return cnt + plsc.all_reduce_population_count(m)[0]

        cnt = lax.fori_loop(0, W // 16, _scan, rem)

        n_full = cnt // SUB
        lax.fori_loop(0, n_full, _chunk, 0)
        for i in range(SUB // 16):
            pend_g[pl.ds(i * 16, 16)] = pend_g[pl.ds(n_full * SUB + i * 16, 16)]
            pend_d[pl.ds(i * 16, 16)] = pend_d[pl.ds(n_full * SUB + i * 16, 16)]
            pend_e[pl.ds(i * 16, 16)] = pend_e[pl.ds(n_full * SUB + i * 16, 16)]
        return cnt - n_full * SUB

    rem = lax.fori_loop(0, NW, _window, jnp.int32(0))

    k_pad = (SUB - (rem % SUB)) % SUB
    for i in range(SUB // 16):
        @pl.when(i * 16 < k_pad)
        def _pad():
            pend_g[pl.ds(rem + i * 16, 16)] = coff + N + i16
            pend_d[pl.ds(rem + i * 16, 16)] = i16
            pend_e[pl.ds(rem + i * 16, 16)] = zeros16

    @pl.when(rem > 0)
    def _tail():
        _chunk(0, 0)

    for k in range(KMAX):
        blk = s + N_SUBCORES * k

        @pl.when(blk < NBLK)
        def _out():
            def _erow(rr, _2):
                r = k * 16 + rr
                inv = 1.0 / (den2[pl.ds(r * 16, 16)] + 1e-16)
                for q in range(F // 16):
                    outb[rr, pl.ds(q * 16, 16)] = (
                        acc[pl.ds(r * F + q * 16, 16)] * inv
                        + bias_v[pl.ds(q * 16, 16)])
                return _2

            lax.fori_loop(0, 16, _erow, 0)
            pltpu.sync_copy(outb, out_hbm.at[pl.ds(c * N + blk * 16, 16)])


def _edge_pass(h_pad, asrc_pad, adst_pad, sd_pad, c16, bias, N, NP, F,
               W, NW):
    BN = (h_pad.shape[0] // NP) * N
    PEND = W + SUB + 16
    mesh = plsc.VectorSubcoreMesh(core_axis_name="c", subcore_axis_name="s")
    body = functools.partial(_edge_body, N, NP, F, W, NW)
    return pl.kernel(
        body,
        out_type=jax.ShapeDtypeStruct((BN, F), jnp.float32),
        mesh=mesh,
        compiler_params=pltpu.CompilerParams(needs_layout_passes=False),
        scratch_types=[
            pltpu.VMEM((NP,), jnp.float32),
            pltpu.VMEM((NP,), jnp.float32),
            pltpu.VMEM((2 * W,), jnp.int32),
            pltpu.VMEM((2 * W,), jnp.int32),
            pltpu.VMEM((PEND,), jnp.int32),
            pltpu.VMEM((PEND,), jnp.int32),
            pltpu.VMEM((PEND,), jnp.float32),
            pltpu.VMEM((SUB, F), jnp.float32),
            pltpu.VMEM((640 * F,), jnp.float32),
            pltpu.VMEM((640 * 16,), jnp.float32),
            pltpu.VMEM((16, F), jnp.float32),
            pltpu.VMEM((16,), jnp.float32),
            pltpu.VMEM((F,), jnp.float32),
            pltpu.SemaphoreType.DMA,
            pltpu.SemaphoreType.DMA,
        ],
    )(h_pad, asrc_pad, adst_pad, sd_pad, c16, bias)


def kernel(x, edge_index, W, att_src, att_dst, bias):
    B, N, F = x.shape
    E = edge_index.shape[1]
    BN = B * N
    NP = N + 16

    h, asrc, adst, ms, md = _project(x.reshape(BN, F), W, att_src, att_dst)

    sent_a = jnp.full((16,), NEG_BIG, jnp.float32)
    sent_h = jnp.zeros((16, F), jnp.float32)
    h_parts, as_parts, ad_parts = [], [], []
    for b in range(B):
        h_parts += [h[b * N:(b + 1) * N], sent_h]
        as_parts += [asrc[b * N:(b + 1) * N, 0], sent_a]
        ad_parts += [adst[b * N:(b + 1) * N, 0], sent_a]
    h_pad = jnp.concatenate(h_parts, axis=0)
    asrc_pad = jnp.concatenate(as_parts)
    adst_pad = jnp.concatenate(ad_parts)

    EN = E + N
    W = 1296
    NW = (EN + W - 1) // W
    ET = NW * W
    P = ET - EN
    loop_ids = jnp.arange(N, dtype=jnp.int32)
    pad_ids = jnp.arange(P, dtype=jnp.int32)
    src_pad = jnp.concatenate([edge_index[0].astype(jnp.int32), loop_ids,
                               N + (pad_ids % 16)])
    dst_pad = jnp.concatenate([edge_index[1].astype(jnp.int32), loop_ids,
                               jnp.full((P,), N, jnp.int32)])
    sd_pad = jnp.concatenate([src_pad.reshape(NW, W), dst_pad.reshape(NW, W)],
                             axis=1).reshape(-1)

    z0 = ms[0, 0] + md[0, 0]
    cb = jnp.where(z0 >= 0.0, z0, 0.2 * z0)
    c16 = jnp.broadcast_to(cb, (16,)).astype(jnp.float32)

    out = _edge_pass(h_pad, asrc_pad, adst_pad, sd_pad, c16, bias,
                     N, NP, F, W, NW)
    return out.reshape(B, N, F)


# speedup vs baseline: 11.1725x; 1.1236x over previous
import functools

import jax
import jax.numpy as jnp
from jax import lax
from jax.experimental import pallas as pl
from jax.experimental.pallas import tpu as pltpu
from jax.experimental.pallas import tpu_sc as plsc

NEG_BIG = -1e30
SUB = 48
N_SUBCORES = 16
N_CORES = 2


def _proj_body(x_ref, w_ref, asv_ref, adv_ref, h_ref, as_ref, ad_ref, ms_ref, md_ref):
    i = pl.program_id(0)
    h = jnp.dot(x_ref[...], w_ref[...], preferred_element_type=jnp.float32)
    h_ref[...] = h
    a_s = jnp.sum(h * asv_ref[...], axis=1, keepdims=True)
    a_d = jnp.sum(h * adv_ref[...], axis=1, keepdims=True)
    as_ref[...] = a_s
    ad_ref[...] = a_d
    bs = jnp.max(a_s, keepdims=True)
    bd = jnp.max(a_d, keepdims=True)
    ms_ref[...] = jnp.where(i == 0, bs, jnp.maximum(ms_ref[...], bs))
    md_ref[...] = jnp.where(i == 0, bd, jnp.maximum(md_ref[...], bd))


def _project(xr, W, att_src, att_dst):
    BN, F = xr.shape
    RB = 1000
    grid = BN // RB
    return pl.pallas_call(
        _proj_body,
        grid=(grid,),
        in_specs=[
            pl.BlockSpec((RB, F), lambda i: (i, 0)),
            pl.BlockSpec((F, F), lambda i: (0, 0)),
            pl.BlockSpec((1, F), lambda i: (0, 0)),
            pl.BlockSpec((1, F), lambda i: (0, 0)),
        ],
        out_specs=[
            pl.BlockSpec((RB, F), lambda i: (i, 0)),
            pl.BlockSpec((RB, 1), lambda i: (i, 0)),
            pl.BlockSpec((RB, 1), lambda i: (i, 0)),
            pl.BlockSpec((1, 1), lambda i: (0, 0)),
            pl.BlockSpec((1, 1), lambda i: (0, 0)),
        ],
        out_shape=[
            jax.ShapeDtypeStruct((BN, F), jnp.float32),
            jax.ShapeDtypeStruct((BN, 1), jnp.float32),
            jax.ShapeDtypeStruct((BN, 1), jnp.float32),
            jax.ShapeDtypeStruct((1, 1), jnp.float32),
            jax.ShapeDtypeStruct((1, 1), jnp.float32),
        ],
    )(xr, W, att_src.reshape(1, F), att_dst.reshape(1, F))


def _edge_body(N, NP, F, W, NW,
               h_hbm, asrc_hbm, adst_hbm, sd_hbm, c16_hbm, bias_hbm,
               out_hbm,
               asrc_v, adst_v, sdw, sdw2, pend_g, pend_d, pend_e, hrows,
               acc, den_sm, n_full_ref, outb, cbuf, bias_v, sem, semS):
    c = lax.axis_index("c")
    s = lax.axis_index("s")
    coff = c * NP
    NBLK = N // 16
    KMAX = (NBLK + N_SUBCORES - 1) // N_SUBCORES
    SUBC = SUB // 16

    pltpu.sync_copy(asrc_hbm.at[pl.ds(coff * 1, NP)], asrc_v)
    pltpu.sync_copy(adst_hbm.at[pl.ds(coff * 1, NP)], adst_v)
    pltpu.sync_copy(c16_hbm, cbuf)
    pltpu.sync_copy(bias_hbm, bias_v)
    c16 = cbuf[...]

    zeros16 = jnp.zeros((16,), jnp.float32)

    def _zacc(r, carry):
        for q in range(F // 16):
            acc[pl.ds(r * F + q * 16, 16)] = zeros16
        den_sm[r] = 0.0
        return carry

    lax.fori_loop(0, 16 * KMAX, _zacc, 0)

    i16 = lax.iota(jnp.int32, 16)

    def _fire_sd(w_):
        pltpu.async_copy(sd_hbm.at[pl.ds(w_ * (2 * W), 2 * W)], sdw2, semS)

    _fire_sd(0)

    def _fire(k, rbase):
        pltpu.async_copy(h_hbm.at[pend_g.at[pl.ds(k * SUB, SUB)]],
                         hrows.at[pl.ds(rbase, SUB)], sem)

    def _drain1():
        pltpu.make_async_copy(h_hbm.at[pl.ds(0, SUB)],
                              hrows.at[pl.ds(0, SUB)], sem).wait()

    def _proc(k, rbase):
        def _grp(g, _3):
            pd16 = pend_d[pl.ds(k * SUB + g * 16, 16)]
            pe16 = pend_e[pl.ds(k * SUB + g * 16, 16)]
            for rr in range(16):
                d_s = pd16[rr]
                e_b = pe16[jnp.full((16,), rr, jnp.int32)]
                e_s = pe16[rr]
                for q in range(F // 16):
                    acc[pl.ds(d_s * F + q * 16, 16)] = (
                        acc[pl.ds(d_s * F + q * 16, 16)]
                        + hrows[rbase + g * 16 + rr, pl.ds(q * 16, 16)] * e_b)
                den_sm[d_s] = den_sm[d_s] + e_s
            return _3

        lax.fori_loop(0, SUBC, _grp, 0)

    def _group(g, _2):
        k0 = 2 * g
        nf = n_full_ref[0]
        _fire(k0, 0)

        @pl.when(k0 + 1 < nf)
        def _f2():
            _fire(k0 + 1, SUB)

        def _sub(i, _3):
            _drain1()
            _proc(k0 + i, i * SUB)
            return _3

        lax.fori_loop(0, jnp.minimum(2, nf - k0), _sub, 0)
        return _2

    def _window(w, rem):
        pltpu.make_async_copy(sd_hbm.at[pl.ds(0, 2 * W)], sdw2, semS).wait()

        def _cp(j, carry):
            sdw[pl.ds(j * 16, 16)] = sdw2[pl.ds(j * 16, 16)]
            return carry

        lax.fori_loop(0, (2 * W) // 16, _cp, 0)

        @pl.when(w + 1 < NW)
        def _pf():
            _fire_sd(w + 1)

        def _scan(j, cnt):
            s16 = sdw[pl.ds(j * 16, 16)]
            d16 = sdw[pl.ds(W + j * 16, 16)]
            a_s = plsc.load_gather(asrc_v, [s16])
            a_d = plsc.load_gather(adst_v, [d16])
            z = a_s + a_d
            al = jnp.where(z >= 0.0, z, z * 0.2)
            e = jnp.exp(al - c16)
            blk = d16 >> 4
            m = (blk & 15) == s
            d_l = ((d16 >> 8) << 4) | (d16 & 15)
            plsc.store_compressed(pend_g.at[pl.ds(cnt, 16)], s16 + coff, mask=m)
            plsc.store_compressed(pend_d.at[pl.ds(cnt, 16)], d_l, mask=m)
            plsc.store_compressed(pend_e.at[pl.ds(cnt, 16)], e, mask=m)
            return cnt + plsc.all_reduce_population_count(m)[0]

        cnt = lax.fori_loop(0, W // 16, _scan, rem)

        n_full = cnt // SUB
        n_full_ref[0] = n_full
        lax.fori_loop(0, (n_full + 1) // 2, _group, 0)
        for i in range(SUB // 16):
            pend_g[pl.ds(i * 16, 16)] = pend_g[pl.ds(n_full * SUB + i * 16, 16)]
            pend_d[pl.ds(i * 16, 16)] = pend_d[pl.ds(n_full * SUB + i * 16, 16)]
            pend_e[pl.ds(i * 16, 16)] = pend_e[pl.ds(n_full * SUB + i * 16, 16)]
        return cnt - n_full * SUB

    rem = lax.fori_loop(0, NW, _window, jnp.int32(0))

    k_pad = (SUB - (rem % SUB)) % SUB
    for i in range(SUB // 16):
        @pl.when(i * 16 < k_pad)
        def _pad():
            pend_g[pl.ds(rem + i * 16, 16)] = coff + N + i16
            pend_d[pl.ds(rem + i * 16, 16)] = i16
            pend_e[pl.ds(rem + i * 16, 16)] = zeros16

    @pl.when(rem > 0)
    def _tail():
        n_full_ref[0] = 1
        _group(0, 0)

    def _oblk(k, carry):
        blk = s + N_SUBCORES * k

        @pl.when(blk < NBLK)
        def _out():
            def _erow(rr, _2):
                r = k * 16 + rr
                inv = 1.0 / (jnp.broadcast_to(den_sm[r], (16,)) + 1e-16)
                for q in range(F // 16):
                    outb[rr, pl.ds(q * 16, 16)] = (
                        acc[pl.ds(r * F + q * 16, 16)] * inv
                        + bias_v[pl.ds(q * 16, 16)])
                return _2

            lax.fori_loop(0, 16, _erow, 0)
            pltpu.sync_copy(outb, out_hbm.at[pl.ds(c * N + blk * 16, 16)])

        return carry

    lax.fori_loop(0, KMAX, _oblk, 0)


def _edge_pass(h_pad, asrc_pad, adst_pad, sd_pad, c16, bias, N, NP, F,
               W, NW):
    BN = (h_pad.shape[0] // NP) * N
    PEND = W + SUB + 16
    mesh = plsc.VectorSubcoreMesh(core_axis_name="c", subcore_axis_name="s")
    body = functools.partial(_edge_body, N, NP, F, W, NW)
    return pl.kernel(
        body,
        out_type=jax.ShapeDtypeStruct((BN, F), jnp.float32),
        mesh=mesh,
        compiler_params=pltpu.CompilerParams(needs_layout_passes=False),
        scratch_types=[
            pltpu.VMEM((NP,), jnp.float32),
            pltpu.VMEM((NP,), jnp.float32),
            pltpu.VMEM((2 * W,), jnp.int32),
            pltpu.VMEM((2 * W,), jnp.int32),
            pltpu.VMEM((PEND,), jnp.int32),
            pltpu.VMEM((PEND,), jnp.int32),
            pltpu.VMEM((PEND,), jnp.float32),
            pltpu.VMEM((2 * SUB, F), jnp.float32),
            pltpu.VMEM((640 * F,), jnp.float32),
            pltpu.SMEM((640,), jnp.float32),
            pltpu.SMEM((1,), jnp.int32),
            pltpu.VMEM((16, F), jnp.float32),
            pltpu.VMEM((16,), jnp.float32),
            pltpu.VMEM((F,), jnp.float32),
            pltpu.SemaphoreType.DMA,
            pltpu.SemaphoreType.DMA,
        ],
    )(h_pad, asrc_pad, adst_pad, sd_pad, c16, bias)


def kernel(x, edge_index, W, att_src, att_dst, bias):
    B, N, F = x.shape
    E = edge_index.shape[1]
    BN = B * N
    NP = N + 16

    h, asrc, adst, ms, md = _project(x.reshape(BN, F), W, att_src, att_dst)

    sent_a = jnp.full((16,), NEG_BIG, jnp.float32)
    sent_h = jnp.zeros((16, F), jnp.float32)
    h_parts, as_parts, ad_parts = [], [], []
    for b in range(B):
        h_parts += [h[b * N:(b + 1) * N], sent_h]
        as_parts += [asrc[b * N:(b + 1) * N, 0], sent_a]
        ad_parts += [adst[b * N:(b + 1) * N, 0], sent_a]
    h_pad = jnp.concatenate(h_parts, axis=0)
    asrc_pad = jnp.concatenate(as_parts)
    adst_pad = jnp.concatenate(ad_parts)

    EN = E + N
    W = 1296
    NW = (EN + W - 1) // W
    ET = NW * W
    P = ET - EN
    loop_ids = jnp.arange(N, dtype=jnp.int32)
    pad_ids = jnp.arange(P, dtype=jnp.int32)
    src_pad = jnp.concatenate([edge_index[0].astype(jnp.int32), loop_ids,
                               N + (pad_ids % 16)])
    dst_pad = jnp.concatenate([edge_index[1].astype(jnp.int32), loop_ids,
                               jnp.full((P,), N, jnp.int32)])
    sd_pad = jnp.concatenate([src_pad.reshape(NW, W), dst_pad.reshape(NW, W)],
                             axis=1).reshape(-1)

    z0 = ms[0, 0] + md[0, 0]
    cb = jnp.where(z0 >= 0.0, z0, 0.2 * z0)
    c16 = jnp.broadcast_to(cb, (16,)).astype(jnp.float32)

    out = _edge_pass(h_pad, asrc_pad, adst_pad, sd_pad, c16, bias,
                     N, NP, F, W, NW)
    return out.reshape(B, N, F)
